# baseline (speedup 1.0000x reference)
import functools

import jax
import jax.numpy as jnp
from jax import lax
from jax.experimental import pallas as pl
from jax.experimental.pallas import tpu as pltpu
from jax.experimental.pallas import tpu_sc as plsc

SEQ = 200
DIM = 64
NUM_CORES = 2
NUM_SUBCORES = 16
NUM_WORKERS = NUM_CORES * NUM_SUBCORES
GATHER_CHUNK = 40
NCHUNK = SEQ // GATHER_CHUNK


def _body(table_hbm, idx_hbm, pos_hbm, out_hbm,
          idx_v, gbuf, sbuf, pos_v, gsems, ssems):
    n_seq = idx_hbm.shape[0] // SEQ
    seq_per_w = n_seq // NUM_WORKERS
    wid = lax.axis_index("s") * NUM_CORES + lax.axis_index("c")
    my_base = wid * seq_per_w * SEQ

    pltpu.sync_copy(pos_hbm, pos_v)
    pltpu.sync_copy(idx_hbm.at[pl.ds(my_base, seq_per_w * SEQ)], idx_v)

    def fire_gathers(s, b):
        for j in range(NCHUNK):
            pltpu.async_copy(
                table_hbm.at[idx_v.at[pl.ds(s * SEQ + j * GATHER_CHUNK,
                                            GATHER_CHUNK)]],
                gbuf.at[b, pl.ds(j * GATHER_CHUNK, GATHER_CHUNK), :],
                gsems.at[b],
            )

    def wait_gathers(b):
        for j in range(NCHUNK):
            pltpu.make_async_copy(
                table_hbm.at[idx_v.at[pl.ds(j * GATHER_CHUNK, GATHER_CHUNK)]],
                gbuf.at[b, pl.ds(j * GATHER_CHUNK, GATHER_CHUNK), :],
                gsems.at[b],
            ).wait()

    def wait_store(b):
        pltpu.make_async_copy(sbuf.at[b], out_hbm.at[0], ssems.at[b]).wait()

    for b in range(2):
        fire_gathers(b, b)

    @pl.loop(0, seq_per_w // 2)
    def _pair_loop(i):
        for b in range(2):
            cur = i * 2 + b
            wait_gathers(b)

            @pl.when(i >= 1)
            def _():
                wait_store(b)

            @pl.loop(0, SEQ)
            def _row_loop(r):
                for c in range(DIM // 16):
                    sl = pl.ds(c * 16, 16)
                    sbuf[b, r, sl] = gbuf[b, r, sl] + pos_v[r, sl]

            @pl.when(i < seq_per_w // 2 - 1)
            def _():
                fire_gathers(cur + 2, b)

            pltpu.async_copy(
                sbuf.at[b],
                out_hbm.at[wid * seq_per_w + cur],
                ssems.at[b],
            )

    for b in range(2):
        wait_store(b)


def kernel(inputs, token_table, pos_table):
    batch, seq = inputs.shape
    flat_idx = inputs.reshape(batch * seq)
    seq_per_w = batch // NUM_WORKERS
    mesh = plsc.VectorSubcoreMesh(
        core_axis_name="c",
        subcore_axis_name="s",
        num_cores=NUM_CORES,
        num_subcores=NUM_SUBCORES,
    )
    out = pl.kernel(
        _body,
        out_type=jax.ShapeDtypeStruct((batch, seq, DIM), jnp.float32),
        mesh=mesh,
        scratch_types=[
            pltpu.VMEM((seq_per_w * SEQ,), jnp.int32),
            pltpu.VMEM((2, SEQ, DIM), jnp.float32),
            pltpu.VMEM((2, SEQ, DIM), jnp.float32),
            pltpu.VMEM((SEQ, DIM), jnp.float32),
            pltpu.SemaphoreType.DMA((2,)),
            pltpu.SemaphoreType.DMA((2,)),
        ],
        compiler_params=pltpu.CompilerParams(use_tc_tiling_on_sc=False),
    )(token_table, flat_idx, pos_table)
    return out


# speedup vs baseline: 1.1258x; 1.1258x over previous
import functools

import jax
import jax.numpy as jnp
from jax import lax
from jax.experimental import pallas as pl
from jax.experimental.pallas import tpu as pltpu
from jax.experimental.pallas import tpu_sc as plsc

SEQ = 200
DIM = 64
PAD_DIM = 128
NUM_CORES = 2
NUM_SUBCORES = 16
NUM_WORKERS = NUM_CORES * NUM_SUBCORES
CHUNK = 40
NCHUNK = SEQ // CHUNK


def _body(table_hbm, idx_hbm, pos_hbm, out_hbm, idx_v, gbuf, sbuf, pos_v,
          gsems, ssems):
    n_seq = idx_hbm.shape[0] // SEQ
    seq_per_w = n_seq // NUM_WORKERS
    wid = lax.axis_index("s") * NUM_CORES + lax.axis_index("c")
    my_base = wid * seq_per_w * SEQ

    pltpu.sync_copy(pos_hbm, pos_v)
    pltpu.sync_copy(idx_hbm.at[pl.ds(my_base, seq_per_w * SEQ)], idx_v)

    def fire_gather(s, j):
        pltpu.async_copy(
            table_hbm.at[idx_v.at[pl.ds(s * SEQ + j * CHUNK, CHUNK)]],
            gbuf.at[j],
            gsems.at[j],
        )

    def wait_gather(j):
        pltpu.make_async_copy(
            table_hbm.at[idx_v.at[pl.ds(j * CHUNK, CHUNK)]],
            gbuf.at[j],
            gsems.at[j],
        ).wait()

    def wait_store(j):
        pltpu.make_async_copy(
            sbuf.at[j], out_hbm.at[0, pl.ds(j * CHUNK, CHUNK), :], ssems.at[j],
        ).wait()

    for j in range(NCHUNK):
        fire_gather(0, j)

    @pl.loop(0, seq_per_w)
    def _seq_loop(s):
        for j in range(NCHUNK):
            wait_gather(j)

            @pl.when(s >= 1)
            def _():
                wait_store(j)

            @pl.loop(0, CHUNK)
            def _row_loop(r):
                for c in range(DIM // 16):
                    sl = pl.ds(c * 16, 16)
                    sbuf[j, r, sl] = gbuf[j, r, sl] + pos_v[j * CHUNK + r, sl]

            @pl.when(s < seq_per_w - 1)
            def _():
                fire_gather(s + 1, j)

            pltpu.async_copy(
                sbuf.at[j],
                out_hbm.at[wid * seq_per_w + s, pl.ds(j * CHUNK, CHUNK), :],
                ssems.at[j],
            )

    for j in range(NCHUNK):
        wait_store(j)


def kernel(inputs, token_table, pos_table):
    batch, seq = inputs.shape
    flat_idx = inputs.reshape(batch * seq)
    table_pad = jnp.pad(token_table, ((0, 0), (0, PAD_DIM - DIM)))
    pos_pad = jnp.pad(pos_table, ((0, 0), (0, PAD_DIM - DIM)))
    mesh = plsc.VectorSubcoreMesh(
        core_axis_name="c",
        subcore_axis_name="s",
        num_cores=NUM_CORES,
        num_subcores=NUM_SUBCORES,
    )
    seq_per_w = batch // NUM_WORKERS
    out = pl.kernel(
        _body,
        out_type=jax.ShapeDtypeStruct((batch, seq, DIM), jnp.float32),
        mesh=mesh,
        scratch_types=[
            pltpu.VMEM((seq_per_w * SEQ,), jnp.int32),
            pltpu.VMEM((NCHUNK, CHUNK, PAD_DIM), jnp.float32),
            pltpu.VMEM((NCHUNK, CHUNK, DIM), jnp.float32),
            pltpu.VMEM((SEQ, PAD_DIM), jnp.float32),
            pltpu.SemaphoreType.DMA((NCHUNK,)),
            pltpu.SemaphoreType.DMA((NCHUNK,)),
        ],
        compiler_params=pltpu.CompilerParams(use_tc_tiling_on_sc=True),
    )(table_pad, flat_idx, pos_pad)
    return out
